# baseline (speedup 1.0000x reference)
import functools

import jax
import jax.numpy as jnp
from jax.experimental import pallas as pl
from jax.experimental.pallas import tpu as pltpu

TOPK = 8
DOC_LEN = 100
N_KEYS = 100000
D = 128
Q = 64

BLK = 2048
NBLK = (N_KEYS + BLK - 1) // BLK

_NEG_INF = float("-inf")
_BIG_I32 = 2**30


def _topk_kernel(start_ref, q_ref, e_ref, vals_out, idx_out, rv_ref, ri_ref):
    i = pl.program_id(0)

    @pl.when(i == 0)
    def _init():
        rv_ref[...] = jnp.full((Q, TOPK), _NEG_INF, jnp.float32)
        ri_ref[...] = jnp.zeros((Q, TOPK), jnp.int32)

    q = q_ref[...]
    denom = jnp.clip(jnp.sum(jnp.abs(q), axis=1, keepdims=True), 1e-12, None)
    qn = q / denom

    s = jax.lax.dot_general(
        qn, e_ref[...], (((1,), (0,)), ((), ())),
        preferred_element_type=jnp.float32,
    )

    start = start_ref[0]
    col = i * BLK + jax.lax.broadcasted_iota(jnp.int32, (Q, BLK), 1)
    valid = (col < N_KEYS) & ((col < start) | (col >= start + DOC_LEN))
    s = jnp.where(valid, s, _NEG_INF)

    bv, bi = [], []
    for _ in range(TOPK):
        m = jnp.max(s, axis=1, keepdims=True)
        am = jnp.min(jnp.where(s == m, col, _BIG_I32), axis=1, keepdims=True)
        bv.append(m)
        bi.append(am)
        s = jnp.where(col == am, _NEG_INF, s)
    bv = jnp.concatenate(bv, axis=1)
    bi = jnp.concatenate(bi, axis=1)

    cv = jnp.concatenate([rv_ref[...], bv], axis=1)
    ci = jnp.concatenate([ri_ref[...], bi], axis=1)
    nv, ni = [], []
    for _ in range(TOPK):
        m = jnp.max(cv, axis=1, keepdims=True)
        am = jnp.min(jnp.where(cv == m, ci, _BIG_I32), axis=1, keepdims=True)
        nv.append(m)
        ni.append(am)
        cv = jnp.where((cv == m) & (ci == am), _NEG_INF, cv)
    rv_ref[...] = jnp.concatenate(nv, axis=1)
    ri_ref[...] = jnp.concatenate(ni, axis=1)

    @pl.when(i == NBLK - 1)
    def _emit():
        vals_out[...] = rv_ref[...]
        idx_out[...] = ri_ref[...]


@functools.partial(jax.jit, static_argnames=())
def kernel(query, embeddings, doc_id):
    start = (jnp.asarray(doc_id, jnp.int32) * DOC_LEN).reshape((1,))
    grid_spec = pltpu.PrefetchScalarGridSpec(
        num_scalar_prefetch=1,
        grid=(NBLK,),
        in_specs=[
            pl.BlockSpec((Q, D), lambda i, s: (0, 0)),
            pl.BlockSpec((D, BLK), lambda i, s: (0, i)),
        ],
        out_specs=[
            pl.BlockSpec((Q, TOPK), lambda i, s: (0, 0)),
            pl.BlockSpec((Q, TOPK), lambda i, s: (0, 0)),
        ],
        scratch_shapes=[
            pltpu.VMEM((Q, TOPK), jnp.float32),
            pltpu.VMEM((Q, TOPK), jnp.int32),
        ],
    )
    values, indices = pl.pallas_call(
        _topk_kernel,
        grid_spec=grid_spec,
        out_shape=[
            jax.ShapeDtypeStruct((Q, TOPK), jnp.float32),
            jax.ShapeDtypeStruct((Q, TOPK), jnp.int32),
        ],
    )(start, query, embeddings)
    return values, indices


# speedup vs baseline: 2.6488x; 2.6488x over previous
import jax
import jax.numpy as jnp
from jax.experimental import pallas as pl
from jax.experimental.pallas import tpu as pltpu

TOPK = 8
DOC_LEN = 100
N_KEYS = 100000
D = 128
Q = 64

BLK = 2048
SUB = BLK // 128
NBLK = (N_KEYS + BLK - 1) // BLK

_NEG_INF = float("-inf")
_BIG_I32 = 2**30

_SORT8 = [
    (0, 1), (2, 3), (4, 5), (6, 7),
    (0, 2), (1, 3), (4, 6), (5, 7),
    (1, 2), (5, 6),
    (0, 4), (1, 5), (2, 6), (3, 7),
    (2, 4), (3, 5),
    (1, 2), (3, 4), (5, 6),
]
_BMERGE8 = [
    (0, 4), (1, 5), (2, 6), (3, 7),
    (0, 2), (1, 3), (4, 6), (5, 7),
    (0, 1), (2, 3), (4, 5), (6, 7),
]


def _ce(v, x, a, b):
    c = v[a] >= v[b]
    va, vb = jnp.where(c, v[a], v[b]), jnp.where(c, v[b], v[a])
    xa, xb = jnp.where(c, x[a], x[b]), jnp.where(c, x[b], x[a])
    v[a], v[b], x[a], x[b] = va, vb, xa, xb


def _sort8(v, x):
    for a, b in _SORT8:
        _ce(v, x, a, b)


def _bmerge8(v, x):
    for a, b in _BMERGE8:
        _ce(v, x, a, b)


def _top8_merge(av, ax, bv, bx):
    mv, mx = [], []
    for j in range(TOPK):
        c = av[j] >= bv[TOPK - 1 - j]
        mv.append(jnp.where(c, av[j], bv[TOPK - 1 - j]))
        mx.append(jnp.where(c, ax[j], bx[TOPK - 1 - j]))
    _bmerge8(mv, mx)
    return mv, mx


def _topk_kernel(start_ref, q_ref, e_ref, vals_out, idx_out, rv_ref, ri_ref):
    i = pl.program_id(0)

    @pl.when(i == 0)
    def _init():
        rv_ref[...] = jnp.full((Q, TOPK * 128), _NEG_INF, jnp.float32)
        ri_ref[...] = jnp.zeros((Q, TOPK * 128), jnp.int32)

    q = q_ref[...]
    denom = jnp.clip(jnp.sum(jnp.abs(q), axis=1, keepdims=True), 1e-12, None)
    qn = q / denom

    s = jax.lax.dot_general(
        qn, e_ref[...], (((1,), (0,)), ((), ())),
        preferred_element_type=jnp.float32,
    )

    start = start_ref[0]
    col = i * BLK + jax.lax.broadcasted_iota(jnp.int32, (Q, BLK), 1)
    valid = (col < N_KEYS) & ((col < start) | (col >= start + DOC_LEN))
    s = jnp.where(valid, s, _NEG_INF)

    xv = [s[:, c * 128:(c + 1) * 128] for c in range(SUB)]
    xi = [col[:, c * 128:(c + 1) * 128] for c in range(SUB)]

    av, ax = xv[:TOPK], xi[:TOPK]
    bv, bx = xv[TOPK:], xi[TOPK:]
    _sort8(av, ax)
    _sort8(bv, bx)
    nv, nx = _top8_merge(av, ax, bv, bx)

    cv = [rv_ref[:, j * 128:(j + 1) * 128] for j in range(TOPK)]
    cx = [ri_ref[:, j * 128:(j + 1) * 128] for j in range(TOPK)]
    mv, mx = _top8_merge(cv, cx, nv, nx)
    rv_ref[...] = jnp.concatenate(mv, axis=1)
    ri_ref[...] = jnp.concatenate(mx, axis=1)

    @pl.when(i == NBLK - 1)
    def _emit():
        v_all = jnp.concatenate(mv, axis=1)
        i_all = jnp.concatenate(mx, axis=1)
        ov, oi = [], []
        for _ in range(TOPK):
            m = jnp.max(v_all, axis=1, keepdims=True)
            am = jnp.min(jnp.where(v_all == m, i_all, _BIG_I32),
                         axis=1, keepdims=True)
            ov.append(m)
            oi.append(am)
            v_all = jnp.where((v_all == m) & (i_all == am), _NEG_INF, v_all)
        vals_out[...] = jnp.concatenate(ov, axis=1)
        idx_out[...] = jnp.concatenate(oi, axis=1)


def kernel(query, embeddings, doc_id):
    start = (jnp.asarray(doc_id, jnp.int32) * DOC_LEN).reshape((1,))
    grid_spec = pltpu.PrefetchScalarGridSpec(
        num_scalar_prefetch=1,
        grid=(NBLK,),
        in_specs=[
            pl.BlockSpec((Q, D), lambda i, s: (0, 0)),
            pl.BlockSpec((D, BLK), lambda i, s: (0, i)),
        ],
        out_specs=[
            pl.BlockSpec((Q, TOPK), lambda i, s: (0, 0)),
            pl.BlockSpec((Q, TOPK), lambda i, s: (0, 0)),
        ],
        scratch_shapes=[
            pltpu.VMEM((Q, TOPK * 128), jnp.float32),
            pltpu.VMEM((Q, TOPK * 128), jnp.int32),
        ],
    )
    values, indices = pl.pallas_call(
        _topk_kernel,
        grid_spec=grid_spec,
        out_shape=[
            jax.ShapeDtypeStruct((Q, TOPK), jnp.float32),
            jax.ShapeDtypeStruct((Q, TOPK), jnp.int32),
        ],
    )(start, query, embeddings)
    return values, indices


# speedup vs baseline: 3.2280x; 1.2186x over previous
import jax
import jax.numpy as jnp
from jax.experimental import pallas as pl
from jax.experimental.pallas import tpu as pltpu

TOPK = 8
DOC_LEN = 100
N_KEYS = 100000
D = 128
Q = 64

BLK = 8192
SUB = BLK // 128
NGRP = SUB // 8
NBLK = (N_KEYS + BLK - 1) // BLK

_NEG_INF = float("-inf")
_BIG_I32 = 2**30

_SORT8 = [
    (0, 1), (2, 3), (4, 5), (6, 7),
    (0, 2), (1, 3), (4, 6), (5, 7),
    (1, 2), (5, 6),
    (0, 4), (1, 5), (2, 6), (3, 7),
    (2, 4), (3, 5),
    (1, 2), (3, 4), (5, 6),
]
_BMERGE8 = [
    (0, 4), (1, 5), (2, 6), (3, 7),
    (0, 2), (1, 3), (4, 6), (5, 7),
    (0, 1), (2, 3), (4, 5), (6, 7),
]


def _ce(v, x, a, b):
    c = v[a] >= v[b]
    va, vb = jnp.where(c, v[a], v[b]), jnp.where(c, v[b], v[a])
    xa, xb = jnp.where(c, x[a], x[b]), jnp.where(c, x[b], x[a])
    v[a], v[b], x[a], x[b] = va, vb, xa, xb


def _sort8(v, x):
    for a, b in _SORT8:
        _ce(v, x, a, b)


def _top8_merge(av, ax, bv, bx):
    mv, mx = [], []
    for j in range(TOPK):
        c = av[j] >= bv[TOPK - 1 - j]
        mv.append(jnp.where(c, av[j], bv[TOPK - 1 - j]))
        mx.append(jnp.where(c, ax[j], bx[TOPK - 1 - j]))
    for a, b in _BMERGE8:
        c = mv[a] >= mv[b]
        mv[a], mv[b] = jnp.where(c, mv[a], mv[b]), jnp.where(c, mv[b], mv[a])
        mx[a], mx[b] = jnp.where(c, mx[a], mx[b]), jnp.where(c, mx[b], mx[a])
    return mv, mx


def _topk_kernel(start_ref, q_ref, e_ref, vals_out, idx_out, rv_ref, ri_ref):
    i = pl.program_id(0)

    @pl.when(i == 0)
    def _init():
        rv_ref[...] = jnp.full((Q, TOPK * 128), _NEG_INF, jnp.float32)
        ri_ref[...] = jnp.zeros((Q, TOPK * 128), jnp.int32)

    q = q_ref[...]
    denom = jnp.clip(jnp.sum(jnp.abs(q), axis=1, keepdims=True), 1e-12, None)
    qn = q / denom

    s = jax.lax.dot_general(
        qn, e_ref[...], (((1,), (0,)), ((), ())),
        preferred_element_type=jnp.float32,
    )

    start = start_ref[0]
    end = start + DOC_LEN
    lane = jax.lax.broadcasted_iota(jnp.int32, (Q, 128), 1)
    base = i * BLK

    groups = []
    for g in range(NGRP):
        gv, gx = [], []
        for c in range(8):
            off = g * 1024 + c * 128
            col = lane + (base + off)
            x = s[:, off:off + 128]
            valid = (col < N_KEYS) & ((col < start) | (col >= end))
            gv.append(jnp.where(valid, x, _NEG_INF))
            gx.append(col)
        _sort8(gv, gx)
        groups.append((gv, gx))
    while len(groups) > 1:
        groups = [
            _top8_merge(groups[k][0], groups[k][1],
                        groups[k + 1][0], groups[k + 1][1])
            for k in range(0, len(groups), 2)
        ]
    nv, nx = groups[0]

    cv = [rv_ref[:, j * 128:(j + 1) * 128] for j in range(TOPK)]
    cx = [ri_ref[:, j * 128:(j + 1) * 128] for j in range(TOPK)]
    mv, mx = _top8_merge(cv, cx, nv, nx)
    rv_ref[...] = jnp.concatenate(mv, axis=1)
    ri_ref[...] = jnp.concatenate(mx, axis=1)

    @pl.when(i == NBLK - 1)
    def _emit():
        v_all = jnp.concatenate(mv, axis=1)
        i_all = jnp.concatenate(mx, axis=1)
        ov, oi = [], []
        for _ in range(TOPK):
            m = jnp.max(v_all, axis=1, keepdims=True)
            am = jnp.min(jnp.where(v_all == m, i_all, _BIG_I32),
                         axis=1, keepdims=True)
            ov.append(m)
            oi.append(am)
            v_all = jnp.where((v_all == m) & (i_all == am), _NEG_INF, v_all)
        vals_out[...] = jnp.concatenate(ov, axis=1)
        idx_out[...] = jnp.concatenate(oi, axis=1)


def kernel(query, embeddings, doc_id):
    start = (jnp.asarray(doc_id, jnp.int32) * DOC_LEN).reshape((1,))
    grid_spec = pltpu.PrefetchScalarGridSpec(
        num_scalar_prefetch=1,
        grid=(NBLK,),
        in_specs=[
            pl.BlockSpec((Q, D), lambda i, s: (0, 0)),
            pl.BlockSpec((D, BLK), lambda i, s: (0, i)),
        ],
        out_specs=[
            pl.BlockSpec((Q, TOPK), lambda i, s: (0, 0)),
            pl.BlockSpec((Q, TOPK), lambda i, s: (0, 0)),
        ],
        scratch_shapes=[
            pltpu.VMEM((Q, TOPK * 128), jnp.float32),
            pltpu.VMEM((Q, TOPK * 128), jnp.int32),
        ],
    )
    values, indices = pl.pallas_call(
        _topk_kernel,
        grid_spec=grid_spec,
        out_shape=[
            jax.ShapeDtypeStruct((Q, TOPK), jnp.float32),
            jax.ShapeDtypeStruct((Q, TOPK), jnp.int32),
        ],
    )(start, query, embeddings)
    return values, indices


# speedup vs baseline: 6.7949x; 2.1050x over previous
import jax
import jax.numpy as jnp
from jax.experimental import pallas as pl
from jax.experimental.pallas import tpu as pltpu

TOPK = 8
DOC_LEN = 100
N_KEYS = 100000
D = 128
Q = 64

BLK = 8192
SUB = BLK // 128
NGRP = SUB // 8
NBLK = (N_KEYS + BLK - 1) // BLK

_NEG_INF = float("-inf")
_BIG_I32 = 2**30

_SORT8 = [
    (0, 1), (2, 3), (4, 5), (6, 7),
    (0, 2), (1, 3), (4, 6), (5, 7),
    (1, 2), (5, 6),
    (0, 4), (1, 5), (2, 6), (3, 7),
    (2, 4), (3, 5),
    (1, 2), (3, 4), (5, 6),
]
_BMERGE8 = [
    (0, 4), (1, 5), (2, 6), (3, 7),
    (0, 2), (1, 3), (4, 6), (5, 7),
    (0, 1), (2, 3), (4, 5), (6, 7),
]


def _ce(v, x, a, b):
    c = v[a] >= v[b]
    va, vb = jnp.where(c, v[a], v[b]), jnp.where(c, v[b], v[a])
    xa, xb = jnp.where(c, x[a], x[b]), jnp.where(c, x[b], x[a])
    v[a], v[b], x[a], x[b] = va, vb, xa, xb


def _sort8(v, x):
    for a, b in _SORT8:
        _ce(v, x, a, b)


def _top8_merge(av, ax, bv, bx):
    mv, mx = [], []
    for j in range(TOPK):
        c = av[j] >= bv[TOPK - 1 - j]
        mv.append(jnp.where(c, av[j], bv[TOPK - 1 - j]))
        mx.append(jnp.where(c, ax[j], bx[TOPK - 1 - j]))
    for a, b in _BMERGE8:
        c = mv[a] >= mv[b]
        mv[a], mv[b] = jnp.where(c, mv[a], mv[b]), jnp.where(c, mv[b], mv[a])
        mx[a], mx[b] = jnp.where(c, mx[a], mx[b]), jnp.where(c, mx[b], mx[a])
    return mv, mx


def _topk_kernel(start_ref, q_ref, e_ref, vals_out, idx_out, rv_ref, ri_ref):
    i = pl.program_id(0)

    @pl.when(i == 0)
    def _init():
        rv_ref[...] = jnp.full((Q, TOPK * 128), _NEG_INF, jnp.float32)
        ri_ref[...] = jnp.zeros((Q, TOPK * 128), jnp.int32)

    q = q_ref[...]
    denom = jnp.clip(jnp.sum(jnp.abs(q), axis=1, keepdims=True), 1e-12, None)
    qn = q / denom

    s = jax.lax.dot_general(
        qn, e_ref[...], (((1,), (1,)), ((), ())),
        preferred_element_type=jnp.float32,
    )

    start = start_ref[0]
    end = start + DOC_LEN
    lane = jax.lax.broadcasted_iota(jnp.int32, (Q, 128), 1)
    base = i * BLK

    groups = []
    for g in range(NGRP):
        gv, gx = [], []
        for c in range(8):
            off = g * 1024 + c * 128
            col = lane + (base + off)
            x = s[:, off:off + 128]
            valid = (col < N_KEYS) & ((col < start) | (col >= end))
            gv.append(jnp.where(valid, x, _NEG_INF))
            gx.append(col)
        _sort8(gv, gx)
        groups.append((gv, gx))
    while len(groups) > 1:
        groups = [
            _top8_merge(groups[k][0], groups[k][1],
                        groups[k + 1][0], groups[k + 1][1])
            for k in range(0, len(groups), 2)
        ]
    nv, nx = groups[0]

    cv = [rv_ref[:, j * 128:(j + 1) * 128] for j in range(TOPK)]
    cx = [ri_ref[:, j * 128:(j + 1) * 128] for j in range(TOPK)]
    mv, mx = _top8_merge(cv, cx, nv, nx)
    rv_ref[...] = jnp.concatenate(mv, axis=1)
    ri_ref[...] = jnp.concatenate(mx, axis=1)

    @pl.when(i == NBLK - 1)
    def _emit():
        v_all = jnp.concatenate(mv, axis=1)
        i_all = jnp.concatenate(mx, axis=1)
        ov, oi = [], []
        for _ in range(TOPK):
            m = jnp.max(v_all, axis=1, keepdims=True)
            am = jnp.min(jnp.where(v_all == m, i_all, _BIG_I32),
                         axis=1, keepdims=True)
            ov.append(m)
            oi.append(am)
            v_all = jnp.where((v_all == m) & (i_all == am), _NEG_INF, v_all)
        vals_out[...] = jnp.concatenate(ov, axis=1)
        idx_out[...] = jnp.concatenate(oi, axis=1)


def kernel(query, embeddings, doc_id):
    corpus = embeddings.T
    start = (jnp.asarray(doc_id, jnp.int32) * DOC_LEN).reshape((1,))
    grid_spec = pltpu.PrefetchScalarGridSpec(
        num_scalar_prefetch=1,
        grid=(NBLK,),
        in_specs=[
            pl.BlockSpec((Q, D), lambda i, s: (0, 0)),
            pl.BlockSpec((BLK, D), lambda i, s: (i, 0)),
        ],
        out_specs=[
            pl.BlockSpec((Q, TOPK), lambda i, s: (0, 0)),
            pl.BlockSpec((Q, TOPK), lambda i, s: (0, 0)),
        ],
        scratch_shapes=[
            pltpu.VMEM((Q, TOPK * 128), jnp.float32),
            pltpu.VMEM((Q, TOPK * 128), jnp.int32),
        ],
    )
    values, indices = pl.pallas_call(
        _topk_kernel,
        grid_spec=grid_spec,
        out_shape=[
            jax.ShapeDtypeStruct((Q, TOPK), jnp.float32),
            jax.ShapeDtypeStruct((Q, TOPK), jnp.int32),
        ],
    )(start, query, corpus)
    return values, indices


# speedup vs baseline: 6.8823x; 1.0129x over previous
import jax
import jax.numpy as jnp
from jax.experimental import pallas as pl
from jax.experimental.pallas import tpu as pltpu

TOPK = 8
DOC_LEN = 100
N_KEYS = 100000
D = 128
Q = 64

BLK = 16384
SUB = BLK // 128
NGRP = SUB // 8
NBLK = (N_KEYS + BLK - 1) // BLK

_NEG_INF = float("-inf")
_BIG_I32 = 2**30

_SORT8 = [
    (0, 1), (2, 3), (4, 5), (6, 7),
    (0, 2), (1, 3), (4, 6), (5, 7),
    (1, 2), (5, 6),
    (0, 4), (1, 5), (2, 6), (3, 7),
    (2, 4), (3, 5),
    (1, 2), (3, 4), (5, 6),
]
_BMERGE8 = [
    (0, 4), (1, 5), (2, 6), (3, 7),
    (0, 2), (1, 3), (4, 6), (5, 7),
    (0, 1), (2, 3), (4, 5), (6, 7),
]


def _ce(v, x, a, b):
    c = v[a] >= v[b]
    va, vb = jnp.where(c, v[a], v[b]), jnp.where(c, v[b], v[a])
    xa, xb = jnp.where(c, x[a], x[b]), jnp.where(c, x[b], x[a])
    v[a], v[b], x[a], x[b] = va, vb, xa, xb


def _sort8(v, x):
    for a, b in _SORT8:
        _ce(v, x, a, b)


def _top8_merge(av, ax, bv, bx):
    mv, mx = [], []
    for j in range(TOPK):
        c = av[j] >= bv[TOPK - 1 - j]
        mv.append(jnp.where(c, av[j], bv[TOPK - 1 - j]))
        mx.append(jnp.where(c, ax[j], bx[TOPK - 1 - j]))
    for a, b in _BMERGE8:
        c = mv[a] >= mv[b]
        mv[a], mv[b] = jnp.where(c, mv[a], mv[b]), jnp.where(c, mv[b], mv[a])
        mx[a], mx[b] = jnp.where(c, mx[a], mx[b]), jnp.where(c, mx[b], mx[a])
    return mv, mx


def _topk_kernel(start_ref, q_ref, e_ref, vals_out, idx_out, rv_ref, ri_ref):
    i = pl.program_id(0)

    @pl.when(i == 0)
    def _init():
        rv_ref[...] = jnp.full((Q, TOPK * 128), _NEG_INF, jnp.float32)
        ri_ref[...] = jnp.zeros((Q, TOPK * 128), jnp.int32)

    q = q_ref[...]
    denom = jnp.clip(jnp.sum(jnp.abs(q), axis=1, keepdims=True), 1e-12, None)
    qn = q / denom

    s = jax.lax.dot_general(
        qn, e_ref[...], (((1,), (1,)), ((), ())),
        preferred_element_type=jnp.float32,
    )

    start = start_ref[0]
    lane = jax.lax.broadcasted_iota(jnp.int32, (Q, 128), 1)
    base = i * BLK

    groups = []
    for g in range(NGRP):
        gv, gx = [], []
        for c in range(8):
            off = g * 1024 + c * 128
            col = lane + (base + off)
            x = s[:, off:off + 128]
            in_doc = (col - start).astype(jnp.uint32) < DOC_LEN
            oob = lane >= (N_KEYS - base - off)
            gv.append(jnp.where(in_doc | oob, _NEG_INF, x))
            gx.append(col)
        _sort8(gv, gx)
        groups.append((gv, gx))
    while len(groups) > 1:
        groups = [
            _top8_merge(groups[k][0], groups[k][1],
                        groups[k + 1][0], groups[k + 1][1])
            for k in range(0, len(groups), 2)
        ]
    nv, nx = groups[0]

    cv = [rv_ref[:, j * 128:(j + 1) * 128] for j in range(TOPK)]
    cx = [ri_ref[:, j * 128:(j + 1) * 128] for j in range(TOPK)]
    mv, mx = _top8_merge(cv, cx, nv, nx)
    rv_ref[...] = jnp.concatenate(mv, axis=1)
    ri_ref[...] = jnp.concatenate(mx, axis=1)

    @pl.when(i == NBLK - 1)
    def _emit():
        v_all = jnp.concatenate(mv, axis=1)
        i_all = jnp.concatenate(mx, axis=1)
        ov, oi = [], []
        for _ in range(TOPK):
            m = jnp.max(v_all, axis=1, keepdims=True)
            am = jnp.min(jnp.where(v_all == m, i_all, _BIG_I32),
                         axis=1, keepdims=True)
            ov.append(m)
            oi.append(am)
            v_all = jnp.where((v_all == m) & (i_all == am), _NEG_INF, v_all)
        vals_out[...] = jnp.concatenate(ov, axis=1).T
        idx_out[...] = jnp.concatenate(oi, axis=1).T


def kernel(query, embeddings, doc_id):
    corpus = embeddings.T
    start = (jnp.asarray(doc_id, jnp.int32) * DOC_LEN).reshape((1,))
    grid_spec = pltpu.PrefetchScalarGridSpec(
        num_scalar_prefetch=1,
        grid=(NBLK,),
        in_specs=[
            pl.BlockSpec((Q, D), lambda i, s: (0, 0)),
            pl.BlockSpec((BLK, D), lambda i, s: (i, 0)),
        ],
        out_specs=[
            pl.BlockSpec((TOPK, Q), lambda i, s: (0, 0)),
            pl.BlockSpec((TOPK, Q), lambda i, s: (0, 0)),
        ],
        scratch_shapes=[
            pltpu.VMEM((Q, TOPK * 128), jnp.float32),
            pltpu.VMEM((Q, TOPK * 128), jnp.int32),
        ],
    )
    values, indices = pl.pallas_call(
        _topk_kernel,
        grid_spec=grid_spec,
        out_shape=[
            jax.ShapeDtypeStruct((TOPK, Q), jnp.float32),
            jax.ShapeDtypeStruct((TOPK, Q), jnp.int32),
        ],
    )(start, query, corpus)
    return values.T, indices.T


# speedup vs baseline: 7.3418x; 1.0668x over previous
import jax
import jax.numpy as jnp
from jax.experimental import pallas as pl
from jax.experimental.pallas import tpu as pltpu

TOPK = 8
DOC_LEN = 100
N_KEYS = 100000
D = 128
Q = 64

BLK = 8192
SUB = BLK // 128
NGRP = SUB // 8
NBLK = (N_KEYS + BLK - 1) // BLK

_NEG_INF = float("-inf")
_BIG_I32 = 2**30

_SORT8 = [
    (0, 1), (2, 3), (4, 5), (6, 7),
    (0, 2), (1, 3), (4, 6), (5, 7),
    (1, 2), (5, 6),
    (0, 4), (1, 5), (2, 6), (3, 7),
    (2, 4), (3, 5),
    (1, 2), (3, 4), (5, 6),
]
_BMERGE8 = [
    (0, 4), (1, 5), (2, 6), (3, 7),
    (0, 2), (1, 3), (4, 6), (5, 7),
    (0, 1), (2, 3), (4, 5), (6, 7),
]


def _ce(v, x, a, b):
    c = v[a] >= v[b]
    va, vb = jnp.where(c, v[a], v[b]), jnp.where(c, v[b], v[a])
    xa, xb = jnp.where(c, x[a], x[b]), jnp.where(c, x[b], x[a])
    v[a], v[b], x[a], x[b] = va, vb, xa, xb


def _sort8(v, x):
    for a, b in _SORT8:
        _ce(v, x, a, b)


def _top8_merge(av, ax, bv, bx):
    mv, mx = [], []
    for j in range(TOPK):
        c = av[j] >= bv[TOPK - 1 - j]
        mv.append(jnp.where(c, av[j], bv[TOPK - 1 - j]))
        mx.append(jnp.where(c, ax[j], bx[TOPK - 1 - j]))
    for a, b in _BMERGE8:
        c = mv[a] >= mv[b]
        mv[a], mv[b] = jnp.where(c, mv[a], mv[b]), jnp.where(c, mv[b], mv[a])
        mx[a], mx[b] = jnp.where(c, mx[a], mx[b]), jnp.where(c, mx[b], mx[a])
    return mv, mx


def _topk_kernel(start_ref, q_ref, e_ref, vals_out, idx_out, rv_ref, ri_ref):
    i = pl.program_id(0)

    @pl.when(i == 0)
    def _init():
        rv_ref[...] = jnp.full((Q, TOPK * 128), _NEG_INF, jnp.float32)
        ri_ref[...] = jnp.zeros((Q, TOPK * 128), jnp.int32)

    q = q_ref[...]
    denom = jnp.clip(jnp.sum(jnp.abs(q), axis=1, keepdims=True), 1e-12, None)
    qn = q / denom

    s = jax.lax.dot_general(
        qn, e_ref[...], (((1,), (1,)), ((), ())),
        preferred_element_type=jnp.float32,
    )

    start = start_ref[0]
    lane = jax.lax.broadcasted_iota(jnp.int32, (Q, 128), 1)
    base = i * BLK

    groups = []
    for g in range(NGRP):
        gv, gx = [], []
        for c in range(8):
            off = g * 1024 + c * 128
            col = lane + (base + off)
            x = s[:, off:off + 128]
            in_doc = (col - start).astype(jnp.uint32) < DOC_LEN
            oob = lane >= (N_KEYS - base - off)
            gv.append(jnp.where(in_doc | oob, _NEG_INF, x))
            gx.append(col)
        _sort8(gv, gx)
        groups.append((gv, gx))
    while len(groups) > 1:
        groups = [
            _top8_merge(groups[k][0], groups[k][1],
                        groups[k + 1][0], groups[k + 1][1])
            for k in range(0, len(groups), 2)
        ]
    nv, nx = groups[0]

    cv = [rv_ref[:, j * 128:(j + 1) * 128] for j in range(TOPK)]
    cx = [ri_ref[:, j * 128:(j + 1) * 128] for j in range(TOPK)]
    mv, mx = _top8_merge(cv, cx, nv, nx)
    rv_ref[...] = jnp.concatenate(mv, axis=1)
    ri_ref[...] = jnp.concatenate(mx, axis=1)

    @pl.when(i == NBLK - 1)
    def _emit():
        v_all = jnp.concatenate(mv, axis=1)
        i_all = jnp.concatenate(mx, axis=1)
        ov, oi = [], []
        for _ in range(TOPK):
            m = jnp.max(v_all, axis=1, keepdims=True)
            am = jnp.min(jnp.where(v_all == m, i_all, _BIG_I32),
                         axis=1, keepdims=True)
            ov.append(m)
            oi.append(am)
            v_all = jnp.where((v_all == m) & (i_all == am), _NEG_INF, v_all)
        vals_out[...] = jnp.concatenate(ov, axis=1).T
        idx_out[...] = jnp.concatenate(oi, axis=1).T


def kernel(query, embeddings, doc_id):
    corpus = embeddings.T
    start = (jnp.asarray(doc_id, jnp.int32) * DOC_LEN).reshape((1,))
    grid_spec = pltpu.PrefetchScalarGridSpec(
        num_scalar_prefetch=1,
        grid=(NBLK,),
        in_specs=[
            pl.BlockSpec((Q, D), lambda i, s: (0, 0)),
            pl.BlockSpec((BLK, D), lambda i, s: (i, 0)),
        ],
        out_specs=[
            pl.BlockSpec((TOPK, Q), lambda i, s: (0, 0)),
            pl.BlockSpec((TOPK, Q), lambda i, s: (0, 0)),
        ],
        scratch_shapes=[
            pltpu.VMEM((Q, TOPK * 128), jnp.float32),
            pltpu.VMEM((Q, TOPK * 128), jnp.int32),
        ],
    )
    values, indices = pl.pallas_call(
        _topk_kernel,
        grid_spec=grid_spec,
        out_shape=[
            jax.ShapeDtypeStruct((TOPK, Q), jnp.float32),
            jax.ShapeDtypeStruct((TOPK, Q), jnp.int32),
        ],
    )(start, query, corpus)
    return values.T, indices.T
